# baseline (speedup 1.0000x reference)
import jax
import jax.numpy as jnp
from jax import lax
from jax.experimental import pallas as pl
from jax.experimental.pallas import tpu as pltpu
from jax.experimental.pallas import tpu_sc as plsc

D = 64
N = 16384
NC = 2
NS = 16
NW = NC * NS
PER_W = N // NW
CHUNK = 128
NCHUNK = PER_W // CHUNK
PART = D + 16


def _sc_body(ids_hbm, w_hbm, table_hbm, out_hbm,
             idx_v, w_v, rows_v, part_v, gather_v, shared, sem):
    cid = lax.axis_index("c")
    sid = lax.axis_index("s")
    wid = sid * NC + cid

    pltpu.sync_copy(ids_hbm.at[wid], idx_v)
    pltpu.sync_copy(w_hbm.at[wid], w_v)

    copies = [
        pltpu.async_copy(table_hbm.at[idx_v.at[j]],
                         rows_v.at[pl.ds(j * CHUNK, CHUNK)], sem)
        for j in range(NCHUNK)
    ]
    for c in copies:
        c.wait()

    zero = jnp.zeros((16,), jnp.float32)

    def body(c, carry):
        a0, a1, a2, a3 = carry
        w_chunk = w_v[pl.ds(c * 16, 16)]
        base = c * 16
        for j in range(16):
            wsp = lax.gather(
                w_chunk, jnp.full((16, 1), j, jnp.int32),
                lax.GatherDimensionNumbers(offset_dims=(),
                                           collapsed_slice_dims=(0,),
                                           start_index_map=(0,)),
                slice_sizes=(1,),
                mode=lax.GatherScatterMode.PROMISE_IN_BOUNDS)
            i = base + j
            a0 = a0 + rows_v[i, pl.ds(0, 16)] * wsp
            a1 = a1 + rows_v[i, pl.ds(16, 16)] * wsp
            a2 = a2 + rows_v[i, pl.ds(32, 16)] * wsp
            a3 = a3 + rows_v[i, pl.ds(48, 16)] * wsp
        return (a0, a1, a2, a3)

    a0, a1, a2, a3 = lax.fori_loop(0, PER_W // 16, body,
                                   (zero, zero, zero, zero))

    def wbody(c, acc):
        return acc + w_v[pl.ds(c * 16, 16)]

    wacc = lax.fori_loop(0, PER_W // 16, wbody, zero)

    part_v[pl.ds(0, 16)] = a0
    part_v[pl.ds(16, 16)] = a1
    part_v[pl.ds(32, 16)] = a2
    part_v[pl.ds(48, 16)] = a3
    part_v[pl.ds(64, 16)] = wacc

    pltpu.sync_copy(part_v, shared.at[sid])
    plsc.subcore_barrier()

    @pl.when(sid == 0)
    def _():
        pltpu.sync_copy(shared, gather_v)
        for k in range(PART // 16):
            s = zero
            for r in range(NS):
                s = s + gather_v[r, pl.ds(k * 16, 16)]
            part_v[pl.ds(k * 16, 16)] = s
        pltpu.sync_copy(part_v, out_hbm.at[cid])


def kernel(ids, weights, table):
    ids_r = ids.astype(jnp.int32).reshape(NW, NCHUNK, CHUNK)
    w_r = weights.reshape(NW, PER_W)
    mesh = plsc.VectorSubcoreMesh(core_axis_name="c", subcore_axis_name="s")
    part = pl.kernel(
        _sc_body,
        mesh=mesh,
        compiler_params=pltpu.CompilerParams(use_tc_tiling_on_sc=False),
        out_type=jax.ShapeDtypeStruct((NC, PART), jnp.float32),
        scratch_types=[
            pltpu.VMEM((NCHUNK, CHUNK), jnp.int32),
            pltpu.VMEM((PER_W,), jnp.float32),
            pltpu.VMEM((PER_W, D), jnp.float32),
            pltpu.VMEM((PART,), jnp.float32),
            pltpu.VMEM((NS, PART), jnp.float32),
            pltpu.VMEM_SHARED((NS, PART), jnp.float32),
            pltpu.SemaphoreType.DMA,
        ],
    )(ids_r, w_r, table)
    sums = part[:, :D].sum(axis=0)
    wsum = part[:, D:].sum()
    return sums / wsum


# speedup vs baseline: 1.7082x; 1.7082x over previous
import jax
import jax.numpy as jnp
from jax import lax
from jax.experimental import pallas as pl
from jax.experimental.pallas import tpu as pltpu
from jax.experimental.pallas import tpu_sc as plsc

D = 64
N = 16384
NC = 2
NS = 16
NW = NC * NS
PER_W = N // NW
IDR = 4
PART = 128


def _sc_body(ids_hbm, w_hbm, table_hbm, out_hbm,
             idx_v, w_v, rows_v, part_v, gather_v, shared, sem):
    cid = lax.axis_index("c")
    sid = lax.axis_index("s")
    wid = sid * NC + cid

    pltpu.sync_copy(ids_hbm.at[pl.ds(wid * IDR, IDR)], idx_v)
    pltpu.sync_copy(w_hbm.at[wid], w_v)

    def fire(c, carry):
        r = c // 8
        k = c - r * 8
        ids16 = idx_v[r, pl.ds(k * 16, 16)]
        base = c * 16
        for j in range(16):
            rid = ids16[j]
            pltpu.async_copy(table_hbm.at[rid], rows_v.at[base + j], sem)
        return carry

    lax.fori_loop(0, PER_W // 16, fire, 0)

    pltpu.make_async_copy(table_hbm.at[pl.ds(0, PER_W)], rows_v, sem).wait()

    zero = jnp.zeros((16,), jnp.float32)

    def body(c, carry):
        a0, a1, a2, a3 = carry
        r = c // 8
        k = c - r * 8
        w_chunk = w_v[pl.ds(c * 16, 16)]
        base = c * 16
        for j in range(16):
            wsp = lax.gather(
                w_chunk, jnp.full((16, 1), j, jnp.int32),
                lax.GatherDimensionNumbers(offset_dims=(),
                                           collapsed_slice_dims=(0,),
                                           start_index_map=(0,)),
                slice_sizes=(1,),
                mode=lax.GatherScatterMode.PROMISE_IN_BOUNDS)
            i = base + j
            a0 = a0 + rows_v[i, pl.ds(0, 16)] * wsp
            a1 = a1 + rows_v[i, pl.ds(16, 16)] * wsp
            a2 = a2 + rows_v[i, pl.ds(32, 16)] * wsp
            a3 = a3 + rows_v[i, pl.ds(48, 16)] * wsp
        return (a0, a1, a2, a3)

    a0, a1, a2, a3 = lax.fori_loop(0, PER_W // 16, body,
                                   (zero, zero, zero, zero))

    def wbody(c, acc):
        return acc + w_v[pl.ds(c * 16, 16)]

    wacc = lax.fori_loop(0, PER_W // 16, wbody, zero)

    part_v[pl.ds(0, 16)] = a0
    part_v[pl.ds(16, 16)] = a1
    part_v[pl.ds(32, 16)] = a2
    part_v[pl.ds(48, 16)] = a3
    part_v[pl.ds(64, 16)] = wacc
    part_v[pl.ds(80, 16)] = zero
    part_v[pl.ds(96, 16)] = zero
    part_v[pl.ds(112, 16)] = zero

    pltpu.sync_copy(part_v, shared.at[sid])
    plsc.subcore_barrier()

    @pl.when(sid == 0)
    def _():
        pltpu.sync_copy(shared, gather_v)
        for k in range(PART // 16):
            s = zero
            for r in range(NS):
                s = s + gather_v[r, pl.ds(k * 16, 16)]
            part_v[pl.ds(k * 16, 16)] = s
        pltpu.sync_copy(part_v, out_hbm.at[cid])


def kernel(ids, weights, table):
    ids_r = ids.astype(jnp.int32).reshape(NW * IDR, 128)
    w_r = weights.reshape(NW, PER_W)
    mesh = plsc.VectorSubcoreMesh(core_axis_name="c", subcore_axis_name="s")
    part = pl.kernel(
        _sc_body,
        mesh=mesh,
        out_type=jax.ShapeDtypeStruct((NC, PART), jnp.float32),
        scratch_types=[
            pltpu.VMEM((IDR, 128), jnp.int32),
            pltpu.VMEM((PER_W,), jnp.float32),
            pltpu.VMEM((PER_W, D), jnp.float32),
            pltpu.VMEM((PART,), jnp.float32),
            pltpu.VMEM((NS, PART), jnp.float32),
            pltpu.VMEM_SHARED((NS, PART), jnp.float32),
            pltpu.SemaphoreType.DMA,
        ],
    )(ids_r, w_r, table)
    sums = part[:, :D].sum(axis=0)
    wsum = part[:, D:D + 16].sum()
    return sums / wsum
